# baseline (speedup 1.0000x reference)
import functools

import jax
import jax.numpy as jnp
from jax import lax
from jax.experimental import pallas as pl
from jax.experimental.pallas import tpu as pltpu
from jax.experimental.pallas import tpu_sc as plsc

_NW = 32
_L = 16


@functools.cache
def _build(B, T, N, C, K, H):
    d = (N * C) // H
    NP = B * K * H
    PPW = NP // _NW
    AG = (B * K) // _L

    mesh = plsc.VectorSubcoreMesh(core_axis_name="c", subcore_axis_name="s")

    @functools.partial(
        pl.kernel,
        mesh=mesh,
        compiler_params=pltpu.CompilerParams(needs_layout_passes=False),
        out_type=[
            jax.ShapeDtypeStruct((NP, d), jnp.float32),
            jax.ShapeDtypeStruct((B * K, C), jnp.float32),
        ],
        scratch_types=[
            pltpu.VMEM((B * K,), jnp.int32),
            pltpu.VMEM((_L, C), jnp.float32),
            pltpu.SemaphoreType.DMA,
            pltpu.SemaphoreType.DMA,
        ],
    )
    def k(patch_hbm, audio_hbm, rowbase_hbm, out_patch, out_audio,
          idx_v, abuf, g0, asem):
        wid = lax.axis_index("s") * 2 + lax.axis_index("c")
        pltpu.sync_copy(rowbase_hbm, idx_v)
        lanes = lax.iota(jnp.int32, _L)

        @pl.when(wid < AG)
        def _():
            rows = plsc.load_gather(idx_v, [wid * _L + lanes])
            pltpu.async_copy(audio_hbm.at[rows], abuf, asem).wait()
            pltpu.sync_copy(abuf, out_audio.at[pl.ds(wid * _L, _L)])

        handles = []
        for i in range(PPW):
            q = wid * PPW + i
            bk = q >> 1
            h = q & 1
            t2 = plsc.load_gather(idx_v, [lanes * 0 + bk])[0]
            handles.append(pltpu.async_copy(
                patch_hbm.at[pl.ds(t2 * 2 + h, 1)],
                out_patch.at[pl.ds(q, 1)], g0))
        for hnd in handles:
            hnd.wait()

    return k


def kernel(top_k_index_sort, patch_feat, audio_feat):
    B, T, N, C = patch_feat.shape
    K = top_k_index_sort.shape[-1]
    H = 2
    d = (N * C) // H
    idx = top_k_index_sort.reshape(B, K).astype(jnp.int32)
    rowbase = (jnp.arange(B, dtype=jnp.int32)[:, None] * T + idx).reshape(B * K)
    patch_view = patch_feat.reshape(B * T * H, d)
    audio_view = audio_feat.reshape(B * T, C)
    out_p, out_a = _build(B, T, N, C, K, H)(patch_view, audio_view, rowbase)
    return out_p.reshape(B, K, N, C), out_a.reshape(B, K, C)


# speedup vs baseline: 6.5417x; 6.5417x over previous
import functools

import jax
import jax.numpy as jnp
from jax import lax
from jax.experimental import pallas as pl
from jax.experimental.pallas import tpu as pltpu
from jax.experimental.pallas import tpu_sc as plsc

_NW = 32
_L = 16


@functools.cache
def _build(B, T, N, C, K, Q):
    NC = N * C
    QS = NC // Q
    NP = B * K * Q
    PPW = NP // _NW
    AG = (B * K) // _L

    mesh = plsc.VectorSubcoreMesh(core_axis_name="c", subcore_axis_name="s")

    @functools.partial(
        pl.kernel,
        mesh=mesh,
        compiler_params=pltpu.CompilerParams(needs_layout_passes=False),
        out_type=[
            jax.ShapeDtypeStruct((B * K * NC,), jnp.float32),
            jax.ShapeDtypeStruct((B * K, C), jnp.float32),
        ],
        scratch_types=[
            pltpu.VMEM((B * K,), jnp.int32),
            pltpu.VMEM((QS,), jnp.float32),
            pltpu.VMEM((QS,), jnp.float32),
            pltpu.VMEM((_L, C), jnp.float32),
            pltpu.SemaphoreType.DMA,
            pltpu.SemaphoreType.DMA,
            pltpu.SemaphoreType.DMA,
            pltpu.SemaphoreType.DMA,
            pltpu.SemaphoreType.DMA,
        ],
    )
    def k(patch_hbm, audio_hbm, rowbase_hbm, out_patch, out_audio,
          idx_v, buf0, buf1, abuf, g0, g1, s0, s1, asem):
        wid = lax.axis_index("s") * 2 + lax.axis_index("c")
        pltpu.sync_copy(rowbase_hbm, idx_v)
        lanes = lax.iota(jnp.int32, _L)

        @pl.when(wid < AG)
        def _():
            rows = plsc.load_gather(idx_v, [wid * _L + lanes])
            pltpu.async_copy(audio_hbm.at[rows], abuf, asem).wait()
            pltpu.sync_copy(abuf, out_audio.at[pl.ds(wid * _L, _L)])

        bufs = (buf0, buf1)
        gsems = (g0, g1)
        ssems = (s0, s1)
        gh = [None] * PPW
        sh = [None] * PPW

        def issue_gather(j):
            q = wid * PPW + j
            bk = q >> 2
            qt = q & 3
            t2 = plsc.load_gather(idx_v, [lanes * 0 + bk])[0]
            src = (t2 * Q + qt) * QS
            return pltpu.async_copy(
                patch_hbm.at[pl.ds(src, QS)], bufs[j % 2], gsems[j % 2])

        def issue_scatter(j):
            q = wid * PPW + j
            return pltpu.async_copy(
                bufs[j % 2], out_patch.at[pl.ds(q * QS, QS)], ssems[j % 2])

        for j in range(PPW):
            if j >= 2:
                sh[j - 2].wait()
            gh[j] = issue_gather(j)
            if j >= 1:
                gh[j - 1].wait()
                sh[j - 1] = issue_scatter(j - 1)
        gh[PPW - 1].wait()
        sh[PPW - 1] = issue_scatter(PPW - 1)
        if PPW >= 2:
            sh[PPW - 2].wait()
        sh[PPW - 1].wait()

    return k


def kernel(top_k_index_sort, patch_feat, audio_feat):
    B, T, N, C = patch_feat.shape
    K = top_k_index_sort.shape[-1]
    Q = 4
    idx = top_k_index_sort.reshape(B, K).astype(jnp.int32)
    rowbase = (jnp.arange(B, dtype=jnp.int32)[:, None] * T + idx).reshape(B * K)
    patch_view = patch_feat.reshape(B * T * N * C)
    audio_view = audio_feat.reshape(B * T, C)
    out_p, out_a = _build(B, T, N, C, K, Q)(patch_view, audio_view, rowbase)
    return out_p.reshape(B, K, N, C), out_a.reshape(B, K, C)


# speedup vs baseline: 27.1683x; 4.1531x over previous
import functools

import jax
import jax.numpy as jnp
from jax.experimental import pallas as pl
from jax.experimental.pallas import tpu as pltpu


@functools.cache
def _build(B, T, N, C, K):
    def body(idx_ref, patch_ref, audio_ref, outp_ref, outa_ref):
        del idx_ref
        outp_ref[...] = patch_ref[...]
        outa_ref[...] = audio_ref[...]

    grid_spec = pltpu.PrefetchScalarGridSpec(
        num_scalar_prefetch=1,
        grid=(B, K),
        in_specs=[
            pl.BlockSpec((1, 1, N, C), lambda i, j, idx: (i, idx[i, j], 0, 0)),
            pl.BlockSpec((1, 1, 1, C), lambda i, j, idx: (i, idx[i, j], 0, 0)),
        ],
        out_specs=[
            pl.BlockSpec((1, 1, N, C), lambda i, j, idx: (i, j, 0, 0)),
            pl.BlockSpec((1, 1, 1, C), lambda i, j, idx: (i, j, 0, 0)),
        ],
    )
    return pl.pallas_call(
        body,
        grid_spec=grid_spec,
        out_shape=[
            jax.ShapeDtypeStruct((B, K, N, C), jnp.float32),
            jax.ShapeDtypeStruct((B, K, 1, C), jnp.float32),
        ],
    )


def kernel(top_k_index_sort, patch_feat, audio_feat):
    B, T, N, C = patch_feat.shape
    K = top_k_index_sort.shape[-1]
    idx = top_k_index_sort.reshape(B, K).astype(jnp.int32)
    out_p, out_a = _build(B, T, N, C, K)(
        idx, patch_feat, audio_feat.reshape(B, T, 1, C))
    return out_p, out_a.reshape(B, K, C)
